# Initial kernel scaffold; baseline (speedup 1.0000x reference)
import jax
import jax.numpy as jnp
from jax.experimental import pallas as pl


def kernel(x, rows1, cols1, vals1, rows2, cols2, vals2, rows3, cols3, vals3, W1, b1, W2, b2, W3, b3, W4, b4, W5, b5, W6, b6, gamma2, beta2, gamma3, beta3, gamma4, beta4, gamma5, beta5, gamma6, beta6):
    raise NotImplementedError("write your pallas kernel here")


# speedup vs baseline: 132.9771x; 132.9771x over previous
import functools

import jax
import jax.numpy as jnp
from jax import lax
from jax.experimental import pallas as pl
from jax.experimental.pallas import tpu as pltpu
from jax.experimental.pallas import tpu_sc as plsc

_NX1 = [64, 32, 16]
_NX2 = [64, 32, 16]
_NX3 = [6, 6, 6]
_K = 4
_B = 4
_NC = 2
_NS = 16
_NW = _NC * _NS
_DEG = 16
_CN = 8

_GDN = lax.GatherDimensionNumbers(offset_dims=(), collapsed_slice_dims=(0,),
                                  start_index_map=(0,))


def _bcast_lane(v, k):
    idx = jnp.full((16, 1), k, jnp.int32)
    return lax.gather(v, idx, _GDN, (1,),
                      mode=lax.GatherScatterMode.PROMISE_IN_BOUNDS)


@functools.lru_cache(maxsize=None)
def _make_spmv(V, D, alpha, beta):
    NT = V // _NW
    NCH = NT // _CN
    NJ = D // 16
    mesh = plsc.VectorSubcoreMesh(core_axis_name="c", subcore_axis_name="s",
                                  num_cores=_NC, num_subcores=_NS)

    @functools.partial(
        pl.kernel,
        out_type=jax.ShapeDtypeStruct((V, D), jnp.float32),
        mesh=mesh,
        compiler_params=pltpu.CompilerParams(use_tc_tiling_on_sc=False),
        scratch_types=[
            pltpu.VMEM((128,), jnp.int32),
            pltpu.VMEM((128,), jnp.float32),
            pltpu.VMEM((128, D), jnp.float32),
            pltpu.VMEM((_CN, D), jnp.float32),
            pltpu.VMEM((_CN, D), jnp.float32),
            pltpu.VMEM((_CN, D), jnp.float32),
            pltpu.SemaphoreType.DMA,
        ],
    )
    def spmv(xc, xp, cols, vals, out, idx_v, vals_v, gath_v, xc_v, xp_v,
             out_v, sem):
        wid = lax.axis_index("s") * _NC + lax.axis_index("c")
        tile_base = wid * NT

        def chunk_body(ci, carry):
            nb = pl.multiple_of(tile_base + ci * _CN, _CN)
            eb = pl.multiple_of(nb * _DEG, 128)
            pltpu.sync_copy(cols.at[pl.ds(eb, 128)], idx_v)
            pltpu.sync_copy(vals.at[pl.ds(eb, 128)], vals_v)
            gcp = pltpu.async_copy(xc.at[idx_v], gath_v, sem)
            pltpu.sync_copy(xc.at[pl.ds(nb, _CN)], xc_v)
            if beta != 0.0:
                pltpu.sync_copy(xp.at[pl.ds(nb, _CN)], xp_v)
            gcp.wait()

            def node_body(n, c2):
                r0 = n * _DEG
                w16 = vals_v[pl.ds(r0, 16)]
                accs = [jnp.zeros((16,), jnp.float32) for _ in range(NJ)]
                for k in range(_DEG):
                    wk = _bcast_lane(w16, k)
                    for j in range(NJ):
                        accs[j] = accs[j] + wk * gath_v[r0 + k,
                                                        pl.ds(16 * j, 16)]
                for j in range(NJ):
                    res = alpha * (accs[j] - xc_v[n, pl.ds(16 * j, 16)])
                    if beta != 0.0:
                        res = res + beta * xp_v[n, pl.ds(16 * j, 16)]
                    out_v[n, pl.ds(16 * j, 16)] = res
                return c2

            lax.fori_loop(0, _CN, node_body, 0)
            pltpu.sync_copy(out_v, out.at[pl.ds(nb, _CN)])
            return carry

        lax.fori_loop(0, NCH, chunk_body, 0)

    return spmv


def _cheb_states(X0, cols, vals, V, D):
    sp_a = _make_spmv(V, D, 1.0, 0.0)
    sp_b = _make_spmv(V, D, 2.0, -1.0)
    X1 = sp_a(X0, X0, cols, vals)
    X2 = sp_b(X1, X0, cols, vals)
    X3 = sp_b(X2, X1, cols, vals)
    return X0, X1, X2, X3


def _combine(xs, ws, bias):
    y = bias
    for k in range(_K):
        y = y + jnp.dot(xs[k], ws[k], preferred_element_type=jnp.float32,
                        precision=lax.Precision.HIGHEST)
    return jax.nn.relu(y)


_N3 = 6


@functools.lru_cache(maxsize=None)
def _make_dense(V, Din, pool_dims):
    BLK = V // _N3
    pool = pool_dims is not None
    OBLK = BLK // 4 if pool else BLK
    Vo = V // 4 if pool else V

    def body(x0, x1, x2, x3, w0, w1, w2, w3, bias, y_ref, st_ref, acc):
        i = pl.program_id(0)
        y = _combine((x0[...], x1[...], x2[...], x3[...]),
                     (w0[...], w1[...], w2[...], w3[...]), bias[...])
        if pool:
            _, n2, n1 = pool_dims
            y = jnp.max(y.reshape(BLK // 2, 2, 64), axis=1)
            y = jnp.max(y.reshape(n2 // 2, 2, n1 // 2, 64), axis=1)
            y = y.reshape(OBLK, 64)
        y_ref[...] = y
        st = jnp.concatenate([jnp.sum(y, axis=0, keepdims=True),
                              jnp.sum(y * y, axis=0, keepdims=True)], axis=0)

        @pl.when(i == 0)
        def _init():
            acc[...] = st

        @pl.when(i > 0)
        def _accum():
            acc[...] = acc[...] + st

        @pl.when(i == _N3 - 1)
        def _emit():
            st_ref[...] = acc[...]

    xspec = pl.BlockSpec((BLK, Din), lambda i: (i, 0))
    wspec = pl.BlockSpec((Din, 64), lambda i: (0, 0))
    bspec = pl.BlockSpec((1, 64), lambda i: (0, 0))
    return pl.pallas_call(
        body,
        grid=(_N3,),
        in_specs=[xspec] * 4 + [wspec] * 4 + [bspec],
        out_specs=[pl.BlockSpec((OBLK, 64), lambda i: (i, 0)),
                   pl.BlockSpec((2, 64), lambda i: (0, 0))],
        out_shape=[jax.ShapeDtypeStruct((Vo, 64), jnp.float32),
                   jax.ShapeDtypeStruct((2, 64), jnp.float32)],
        scratch_shapes=[pltpu.VMEM((2, 64), jnp.float32)],
    )


@functools.lru_cache(maxsize=None)
def _make_bn_apply(Vo):
    BLK = Vo // _N3

    def body(y, st, gamma, beta, o_ref):
        s, sq = st[0:1, :], st[1:2, :]
        sc = s[:, :16] + s[:, 16:32] + s[:, 32:48] + s[:, 48:]
        sqc = sq[:, :16] + sq[:, 16:32] + sq[:, 32:48] + sq[:, 48:]
        cnt = 4.0 * Vo
        m = sc / cnt
        var = sqc / cnt - m * m
        inv = lax.rsqrt(var + 1e-5)
        mt = jnp.concatenate([m] * 4, axis=1)
        it = jnp.concatenate([inv] * 4, axis=1)
        o_ref[...] = gamma[...] * (y[...] - mt) * it + beta[...]

    return pl.pallas_call(
        body,
        grid=(_N3,),
        in_specs=[pl.BlockSpec((BLK, 64), lambda i: (i, 0)),
                  pl.BlockSpec((2, 64), lambda i: (0, 0)),
                  pl.BlockSpec((1, 64), lambda i: (0, 0)),
                  pl.BlockSpec((1, 64), lambda i: (0, 0))],
        out_specs=pl.BlockSpec((BLK, 64), lambda i: (i, 0)),
        out_shape=jax.ShapeDtypeStruct((Vo, 64), jnp.float32),
    )


def _dense_bn(V, Din, pool_dims, Xs, Wms, bias, gamma, beta):
    y, st = _make_dense(V, Din, pool_dims)(*Xs, *Wms, bias)
    return _make_bn_apply(y.shape[0])(y, st, gamma, beta)


@functools.lru_cache(maxsize=None)
def _make_final(V):
    def body(x0, x1, x2, x3, w0, w1, w2, w3, bias, o_ref):
        y = _combine((x0[...], x1[...], x2[...], x3[...]),
                     (w0[...], w1[...], w2[...], w3[...]), bias[...])
        m = jnp.max(y, axis=0, keepdims=True)
        rows = []
        for b in range(_B):
            sb = m[:, b * 10:(b + 1) * 10]
            mb = jnp.max(sb)
            rows.append(sb - mb - jnp.log(jnp.sum(jnp.exp(sb - mb))))
        o_ref[...] = jnp.concatenate(rows, axis=0)

    return pl.pallas_call(
        body, out_shape=jax.ShapeDtypeStruct((_B, 10), jnp.float32))


def kernel(x, rows1, cols1, vals1, rows2, cols2, vals2, rows3, cols3, vals3,
           W1, b1, W2, b2, W3, b3, W4, b4, W5, b5, W6, b6, gamma2, beta2,
           gamma3, beta3, gamma4, beta4, gamma5, beta5, gamma6, beta6):
    V = [_NX1[i] * _NX2[i] * _NX3[i] for i in range(3)]
    I4 = jnp.eye(_B, dtype=jnp.float32)

    def kron4(w):
        return jnp.kron(I4, w)

    W1m = tuple(jnp.pad(kron4(W1[k]), ((0, 12), (0, 0))) for k in range(_K))
    Wm = {2: W2, 3: W3, 4: W4, 5: W5, 6: W6}
    Wm = {j: tuple(kron4(w[k]) for k in range(_K)) for j, w in Wm.items()}
    biases = {j: jnp.tile(b[None, :], (1, _B))
              for j, b in ((1, b1), (2, b2), (3, b3), (4, b4), (5, b5),
                           (6, b6))}
    gammas = {j: jnp.tile(g[None, :], (1, _B))
              for j, g in ((2, gamma2), (3, gamma3), (4, gamma4),
                           (5, gamma5), (6, gamma6))}
    betas = {j: jnp.tile(b[None, :], (1, _B))
             for j, b in ((2, beta2), (3, beta3), (4, beta4), (5, beta5),
                          (6, beta6))}

    X0 = jnp.pad(x.reshape(_B, V[0]).T, ((0, 0), (0, 12)))

    Xs = _cheb_states(X0, cols1, vals1, V[0], 16)
    h = _dense_bn(V[0], 16, None, Xs, W1m, biases[1], gammas[2], betas[2])
    Xs = _cheb_states(h, cols1, vals1, V[0], 64)
    h = _dense_bn(V[0], 64, (_NX3[0], _NX2[0], _NX1[0]), Xs, Wm[2],
                  biases[2], gammas[3], betas[3])
    Xs = _cheb_states(h, cols2, vals2, V[1], 64)
    h = _dense_bn(V[1], 64, None, Xs, Wm[3], biases[3], gammas[4], betas[4])
    Xs = _cheb_states(h, cols2, vals2, V[1], 64)
    h = _dense_bn(V[1], 64, (_NX3[1], _NX2[1], _NX1[1]), Xs, Wm[4],
                  biases[4], gammas[5], betas[5])
    Xs = _cheb_states(h, cols3, vals3, V[2], 64)
    h = _dense_bn(V[2], 64, None, Xs, Wm[5], biases[5], gammas[6], betas[6])
    Xs = _cheb_states(h, cols3, vals3, V[2], 64)
    return _make_final(V[2])(*Xs, *Wm[6], biases[6])


# speedup vs baseline: 326.5465x; 2.4557x over previous
import functools

import jax
import jax.numpy as jnp
from jax import lax
from jax.experimental import pallas as pl
from jax.experimental.pallas import tpu as pltpu
from jax.experimental.pallas import tpu_sc as plsc

_NX1 = [64, 32, 16]
_NX2 = [64, 32, 16]
_NX3 = [6, 6, 6]
_K = 4
_B = 4
_NC = 2
_NS = 16
_NW = _NC * _NS
_DEG = 16
_CN = 8

_GDN = lax.GatherDimensionNumbers(offset_dims=(), collapsed_slice_dims=(0,),
                                  start_index_map=(0,))


def _bcast_lane(v, k):
    idx = jnp.full((16, 1), k, jnp.int32)
    return lax.gather(v, idx, _GDN, (1,),
                      mode=lax.GatherScatterMode.PROMISE_IN_BOUNDS)


@functools.lru_cache(maxsize=None)
def _make_spmv(V, D, alpha, beta):
    NT = V // _NW
    CN = 32 if NT % 64 == 0 else 24
    G = CN * _DEG // 128
    NCH = NT // CN
    GT = NT // 8
    NJ = D // 16
    mesh = plsc.VectorSubcoreMesh(core_axis_name="c", subcore_axis_name="s",
                                  num_cores=_NC, num_subcores=_NS)

    @functools.partial(
        pl.kernel,
        out_type=jax.ShapeDtypeStruct((V, D), jnp.float32),
        mesh=mesh,
        compiler_params=pltpu.CompilerParams(use_tc_tiling_on_sc=False),
        scratch_types=[
            pltpu.VMEM((GT, 128), jnp.int32),
            pltpu.VMEM((NT * _DEG,), jnp.float32),
            [pltpu.VMEM((CN * _DEG, D), jnp.float32)] * 2,
            [pltpu.VMEM((CN, D), jnp.float32)] * 2,
            [pltpu.VMEM((CN, D), jnp.float32)] * 2,
            [pltpu.VMEM((CN, D), jnp.float32)] * 2,
            [pltpu.SemaphoreType.DMA] * 2,
            [pltpu.SemaphoreType.DMA] * 2,
            [pltpu.SemaphoreType.DMA] * 2,
        ],
    )
    def spmv(xc, xp, cols, vals, out, colv, valv, gath, xcv, xpv, outv,
             sg, sl, so):
        wid = lax.axis_index("s") * _NC + lax.axis_index("c")
        tile_base = wid * NT
        pltpu.sync_copy(cols.at[pl.ds(wid * GT, GT)], colv)
        pltpu.sync_copy(vals.at[pl.ds(tile_base * _DEG, NT * _DEG)], valv)

        def start(ci, b):
            nb = pl.multiple_of(tile_base + ci * CN, 8)
            for g in range(G):
                pltpu.async_copy(xc.at[colv.at[ci * G + g]],
                                 gath[b].at[pl.ds(128 * g, 128)], sg[b])
            pltpu.async_copy(xc.at[pl.ds(nb, CN)], xcv[b], sl[b])
            if beta != 0.0:
                pltpu.async_copy(xp.at[pl.ds(nb, CN)], xpv[b], sl[b])

        def wait_in(b):
            for g in range(G):
                pltpu.make_async_copy(xc.at[colv.at[0]],
                                      gath[b].at[pl.ds(128 * g, 128)],
                                      sg[b]).wait()
            pltpu.make_async_copy(xc.at[pl.ds(0, CN)], xcv[b], sl[b]).wait()
            if beta != 0.0:
                pltpu.make_async_copy(xp.at[pl.ds(0, CN)], xpv[b],
                                      sl[b]).wait()

        def drain_out(b):
            pltpu.make_async_copy(outv[b], out.at[pl.ds(0, CN)],
                                  so[b]).wait()

        def process(ci, b):
            nb = pl.multiple_of(tile_base + ci * CN, 8)
            wait_in(b)

            def node_body(n, c2):
                r0 = n * _DEG
                w16 = valv[pl.ds((ci * CN + n) * _DEG, 16)]
                accs = [jnp.zeros((16,), jnp.float32) for _ in range(NJ)]
                for k in range(_DEG):
                    wk = _bcast_lane(w16, k)
                    for j in range(NJ):
                        accs[j] = accs[j] + wk * gath[b][r0 + k,
                                                        pl.ds(16 * j, 16)]
                for j in range(NJ):
                    res = alpha * (accs[j] - xcv[b][n, pl.ds(16 * j, 16)])
                    if beta != 0.0:
                        res = res + beta * xpv[b][n, pl.ds(16 * j, 16)]
                    outv[b][n, pl.ds(16 * j, 16)] = res
                return c2

            lax.fori_loop(0, CN, node_body, 0)
            pltpu.async_copy(outv[b], out.at[pl.ds(nb, CN)], so[b])

        start(0, 0)
        start(1, 1)
        last = NCH - 1
        process(0, 0)
        start(jnp.minimum(2, last), 0)
        process(1, 1)
        start(jnp.minimum(3, last), 1)

        def outer_body(o, carry):
            for b in (0, 1):
                ci = o * 2 + b
                drain_out(b)
                process(ci, b)
                start(jnp.minimum(ci + 2, last), b)
            return carry

        lax.fori_loop(1, NCH // 2, outer_body, 0)
        for b in (0, 1):
            wait_in(b)
            drain_out(b)

    return spmv


def _cheb_states(X0, cols, vals, V, D):
    cols2d = cols.reshape(V * _DEG // 128, 128)
    sp_a = _make_spmv(V, D, 1.0, 0.0)
    sp_b = _make_spmv(V, D, 2.0, -1.0)
    X1 = sp_a(X0, X0, cols2d, vals)
    X2 = sp_b(X1, X0, cols2d, vals)
    X3 = sp_b(X2, X1, cols2d, vals)
    return X0, X1, X2, X3


def _combine(xs, ws, bias):
    cin = ws[0].shape[0]
    cols = []
    for b in range(_B):
        yb = bias
        for k in range(_K):
            if cin == 1:
                yb = yb + xs[k][:, b:b + 1] * ws[k]
            else:
                yb = yb + jnp.dot(xs[k][:, 16 * b:16 * b + 16], ws[k],
                                  preferred_element_type=jnp.float32)
        cols.append(yb)
    return jax.nn.relu(jnp.concatenate(cols, axis=1))


_N3 = 6


@functools.lru_cache(maxsize=None)
def _make_dense(V, Din, pool_dims):
    BLK = V // _N3
    pool = pool_dims is not None
    OBLK = BLK // 4 if pool else BLK
    Vo = V // 4 if pool else V

    def body(x0, x1, x2, x3, w0, w1, w2, w3, bias, y_ref, st_ref, acc):
        i = pl.program_id(0)
        y = _combine((x0[...], x1[...], x2[...], x3[...]),
                     (w0[...], w1[...], w2[...], w3[...]), bias[...])
        if pool:
            _, n2, n1 = pool_dims
            y = jnp.max(y.reshape(BLK // 2, 2, 64), axis=1)
            y = jnp.max(y.reshape(n2 // 2, 2, n1 // 2, 64), axis=1)
            y = y.reshape(OBLK, 64)
        y_ref[...] = y
        st = jnp.concatenate([jnp.sum(y, axis=0, keepdims=True),
                              jnp.sum(y * y, axis=0, keepdims=True)], axis=0)

        @pl.when(i == 0)
        def _init():
            acc[...] = st

        @pl.when(i > 0)
        def _accum():
            acc[...] = acc[...] + st

        @pl.when(i == _N3 - 1)
        def _emit():
            st_ref[...] = acc[...]

    cin = 1 if Din == 16 else 16
    xspec = pl.BlockSpec((BLK, Din), lambda i: (i, 0))
    wspec = pl.BlockSpec((cin, 16), lambda i: (0, 0))
    bspec = pl.BlockSpec((1, 16), lambda i: (0, 0))
    return pl.pallas_call(
        body,
        grid=(_N3,),
        in_specs=[xspec] * 4 + [wspec] * 4 + [bspec],
        out_specs=[pl.BlockSpec((OBLK, 64), lambda i: (i, 0)),
                   pl.BlockSpec((2, 64), lambda i: (0, 0))],
        out_shape=[jax.ShapeDtypeStruct((Vo, 64), jnp.float32),
                   jax.ShapeDtypeStruct((2, 64), jnp.float32)],
        scratch_shapes=[pltpu.VMEM((2, 64), jnp.float32)],
    )


@functools.lru_cache(maxsize=None)
def _make_bn_apply(Vo):
    BLK = Vo // _N3

    def body(y, st, gamma, beta, o_ref):
        s, sq = st[0:1, :], st[1:2, :]
        sc = s[:, :16] + s[:, 16:32] + s[:, 32:48] + s[:, 48:]
        sqc = sq[:, :16] + sq[:, 16:32] + sq[:, 32:48] + sq[:, 48:]
        cnt = 4.0 * Vo
        m = sc / cnt
        var = sqc / cnt - m * m
        inv = lax.rsqrt(var + 1e-5)
        mt = jnp.concatenate([m] * 4, axis=1)
        it = jnp.concatenate([inv] * 4, axis=1)
        o_ref[...] = gamma[...] * (y[...] - mt) * it + beta[...]

    return pl.pallas_call(
        body,
        grid=(_N3,),
        in_specs=[pl.BlockSpec((BLK, 64), lambda i: (i, 0)),
                  pl.BlockSpec((2, 64), lambda i: (0, 0)),
                  pl.BlockSpec((1, 64), lambda i: (0, 0)),
                  pl.BlockSpec((1, 64), lambda i: (0, 0))],
        out_specs=pl.BlockSpec((BLK, 64), lambda i: (i, 0)),
        out_shape=jax.ShapeDtypeStruct((Vo, 64), jnp.float32),
    )


def _dense_bn(V, Din, pool_dims, Xs, Wms, bias, gamma, beta):
    y, st = _make_dense(V, Din, pool_dims)(*Xs, *Wms, bias)
    return _make_bn_apply(y.shape[0])(y, st, gamma, beta)


@functools.lru_cache(maxsize=None)
def _make_final(V):
    def body(x0, x1, x2, x3, w0, w1, w2, w3, bias, o_ref):
        y = _combine((x0[...], x1[...], x2[...], x3[...]),
                     (w0[...], w1[...], w2[...], w3[...]), bias[...])
        m = jnp.max(y, axis=0, keepdims=True)
        rows = []
        for b in range(_B):
            sb = m[:, b * 10:(b + 1) * 10]
            mb = jnp.max(sb)
            rows.append(sb - mb - jnp.log(jnp.sum(jnp.exp(sb - mb))))
        o_ref[...] = jnp.concatenate(rows, axis=0)

    return pl.pallas_call(
        body, out_shape=jax.ShapeDtypeStruct((_B, 10), jnp.float32))


def kernel(x, rows1, cols1, vals1, rows2, cols2, vals2, rows3, cols3, vals3,
           W1, b1, W2, b2, W3, b3, W4, b4, W5, b5, W6, b6, gamma2, beta2,
           gamma3, beta3, gamma4, beta4, gamma5, beta5, gamma6, beta6):
    V = [_NX1[i] * _NX2[i] * _NX3[i] for i in range(3)]

    W1m = tuple(W1[k] for k in range(_K))
    Wm = {2: W2, 3: W3, 4: W4, 5: W5, 6: W6}
    Wm = {j: tuple(w[k] for k in range(_K)) for j, w in Wm.items()}
    biases = {j: b[None, :]
              for j, b in ((1, b1), (2, b2), (3, b3), (4, b4), (5, b5),
                           (6, b6))}
    gammas = {j: jnp.tile(g[None, :], (1, _B))
              for j, g in ((2, gamma2), (3, gamma3), (4, gamma4),
                           (5, gamma5), (6, gamma6))}
    betas = {j: jnp.tile(b[None, :], (1, _B))
             for j, b in ((2, beta2), (3, beta3), (4, beta4), (5, beta5),
                          (6, beta6))}

    X0 = jnp.pad(x.reshape(_B, V[0]).T, ((0, 0), (0, 12)))

    Xs = _cheb_states(X0, cols1, vals1, V[0], 16)
    h = _dense_bn(V[0], 16, None, Xs, W1m, biases[1], gammas[2], betas[2])
    Xs = _cheb_states(h, cols1, vals1, V[0], 64)
    h = _dense_bn(V[0], 64, (_NX3[0], _NX2[0], _NX1[0]), Xs, Wm[2],
                  biases[2], gammas[3], betas[3])
    Xs = _cheb_states(h, cols2, vals2, V[1], 64)
    h = _dense_bn(V[1], 64, None, Xs, Wm[3], biases[3], gammas[4], betas[4])
    Xs = _cheb_states(h, cols2, vals2, V[1], 64)
    h = _dense_bn(V[1], 64, (_NX3[1], _NX2[1], _NX1[1]), Xs, Wm[4],
                  biases[4], gammas[5], betas[5])
    Xs = _cheb_states(h, cols3, vals3, V[2], 64)
    h = _dense_bn(V[2], 64, None, Xs, Wm[5], biases[5], gammas[6], betas[6])
    Xs = _cheb_states(h, cols3, vals3, V[2], 64)
    return _make_final(V[2])(*Xs, *Wm[6], biases[6])


# speedup vs baseline: 329.1292x; 1.0079x over previous
import functools

import jax
import jax.numpy as jnp
from jax import lax
from jax.experimental import pallas as pl
from jax.experimental.pallas import tpu as pltpu
from jax.experimental.pallas import tpu_sc as plsc

_NX1 = [64, 32, 16]
_NX2 = [64, 32, 16]
_NX3 = [6, 6, 6]
_K = 4
_B = 4
_NC = 2
_NS = 16
_NW = _NC * _NS
_DEG = 16
_CN = 8

_GDN = lax.GatherDimensionNumbers(offset_dims=(), collapsed_slice_dims=(0,),
                                  start_index_map=(0,))


def _bcast_lane(v, k):
    idx = jnp.full((16, 1), k, jnp.int32)
    return lax.gather(v, idx, _GDN, (1,),
                      mode=lax.GatherScatterMode.PROMISE_IN_BOUNDS)


@functools.lru_cache(maxsize=None)
def _make_spmv(V, D, alpha, beta):
    NT = V // _NW
    if D == 16 and NT % 128 == 0:
        CN = 64
    else:
        CN = 32 if NT % 64 == 0 else 24
    G = CN * _DEG // 128
    NCH = NT // CN
    GT = NT // 8
    NJ = D // 16
    mesh = plsc.VectorSubcoreMesh(core_axis_name="c", subcore_axis_name="s",
                                  num_cores=_NC, num_subcores=_NS)

    @functools.partial(
        pl.kernel,
        out_type=jax.ShapeDtypeStruct((V, D), jnp.float32),
        mesh=mesh,
        compiler_params=pltpu.CompilerParams(use_tc_tiling_on_sc=False),
        scratch_types=[
            pltpu.VMEM((GT, 128), jnp.int32),
            pltpu.VMEM((NT * _DEG,), jnp.float32),
            [pltpu.VMEM((CN * _DEG, D), jnp.float32)] * 2,
            [pltpu.VMEM((CN, D), jnp.float32)] * 2,
            [pltpu.VMEM((CN, D), jnp.float32)] * 2,
            [pltpu.VMEM((CN, D), jnp.float32)] * 2,
            [pltpu.SemaphoreType.DMA] * 2,
            [pltpu.SemaphoreType.DMA] * 2,
            [pltpu.SemaphoreType.DMA] * 2,
        ],
    )
    def spmv(xc, xp, cols, vals, out, colv, valv, gath, xcv, xpv, outv,
             sg, sl, so):
        wid = lax.axis_index("s") * _NC + lax.axis_index("c")
        tile_base = wid * NT
        pltpu.sync_copy(cols.at[pl.ds(wid * GT, GT)], colv)
        pltpu.sync_copy(vals.at[pl.ds(tile_base * _DEG, NT * _DEG)], valv)

        def start(ci, b):
            nb = pl.multiple_of(tile_base + ci * CN, 8)
            for g in range(G):
                pltpu.async_copy(xc.at[colv.at[ci * G + g]],
                                 gath[b].at[pl.ds(128 * g, 128)], sg[b])
            pltpu.async_copy(xc.at[pl.ds(nb, CN)], xcv[b], sl[b])
            if beta != 0.0:
                pltpu.async_copy(xp.at[pl.ds(nb, CN)], xpv[b], sl[b])

        def wait_in(b):
            pltpu.make_async_copy(xc.at[colv.at[0]], gath[b], sg[b]).wait()
            pltpu.make_async_copy(xc.at[pl.ds(0, CN)], xcv[b], sl[b]).wait()
            if beta != 0.0:
                pltpu.make_async_copy(xp.at[pl.ds(0, CN)], xpv[b],
                                      sl[b]).wait()

        def drain_out(b):
            pltpu.make_async_copy(outv[b], out.at[pl.ds(0, CN)],
                                  so[b]).wait()

        def process(ci, b):
            nb = pl.multiple_of(tile_base + ci * CN, 8)
            wait_in(b)

            def node_body(n, c2):
                r0 = n * _DEG
                w16 = valv[pl.ds((ci * CN + n) * _DEG, 16)]
                accs = [jnp.zeros((16,), jnp.float32) for _ in range(NJ)]
                for k in range(_DEG):
                    wk = _bcast_lane(w16, k)
                    for j in range(NJ):
                        accs[j] = accs[j] + wk * gath[b][r0 + k,
                                                        pl.ds(16 * j, 16)]
                for j in range(NJ):
                    res = alpha * (accs[j] - xcv[b][n, pl.ds(16 * j, 16)])
                    if beta != 0.0:
                        res = res + beta * xpv[b][n, pl.ds(16 * j, 16)]
                    outv[b][n, pl.ds(16 * j, 16)] = res
                return c2

            lax.fori_loop(0, CN, node_body, 0)
            pltpu.async_copy(outv[b], out.at[pl.ds(nb, CN)], so[b])

        start(0, 0)
        start(1, 1)
        last = NCH - 1
        process(0, 0)
        start(jnp.minimum(2, last), 0)
        process(1, 1)
        start(jnp.minimum(3, last), 1)

        def outer_body(o, carry):
            for b in (0, 1):
                ci = o * 2 + b
                drain_out(b)
                process(ci, b)
                start(jnp.minimum(ci + 2, last), b)
            return carry

        lax.fori_loop(1, NCH // 2, outer_body, 0)
        for b in (0, 1):
            wait_in(b)
            drain_out(b)

    return spmv


def _cheb_states(X0, cols, vals, V, D):
    cols2d = cols.reshape(V * _DEG // 128, 128)
    sp_a = _make_spmv(V, D, 1.0, 0.0)
    sp_b = _make_spmv(V, D, 2.0, -1.0)
    X1 = sp_a(X0, X0, cols2d, vals)
    X2 = sp_b(X1, X0, cols2d, vals)
    X3 = sp_b(X2, X1, cols2d, vals)
    return X0, X1, X2, X3


def _combine(xs, ws, bias):
    cin = ws[0].shape[0]
    cols = []
    if cin == 1:
        for b in range(_B):
            yb = bias
            for k in range(_K):
                yb = yb + xs[k][:, b:b + 1] * ws[k]
            cols.append(yb)
    else:
        wcat = jnp.concatenate(list(ws), axis=0)
        for b in range(_B):
            xb = jnp.concatenate([xs[k][:, 16 * b:16 * b + 16]
                                  for k in range(_K)], axis=1)
            cols.append(bias + jnp.dot(xb, wcat,
                                       preferred_element_type=jnp.float32))
    return jax.nn.relu(jnp.concatenate(cols, axis=1))


_N3 = 6


@functools.lru_cache(maxsize=None)
def _make_dense(V, Din, pool_dims):
    BLK = V // _N3
    pool = pool_dims is not None
    OBLK = BLK // 4 if pool else BLK
    Vo = V // 4 if pool else V

    def body(x0, x1, x2, x3, w0, w1, w2, w3, bias, y_ref, st_ref, acc):
        i = pl.program_id(0)
        y = _combine((x0[...], x1[...], x2[...], x3[...]),
                     (w0[...], w1[...], w2[...], w3[...]), bias[...])
        if pool:
            _, n2, n1 = pool_dims
            y = jnp.max(y.reshape(BLK // 2, 2, 64), axis=1)
            y = jnp.max(y.reshape(n2 // 2, 2, n1 // 2, 64), axis=1)
            y = y.reshape(OBLK, 64)
        y_ref[...] = y
        st = jnp.concatenate([jnp.sum(y, axis=0, keepdims=True),
                              jnp.sum(y * y, axis=0, keepdims=True)], axis=0)

        @pl.when(i == 0)
        def _init():
            acc[...] = st

        @pl.when(i > 0)
        def _accum():
            acc[...] = acc[...] + st

        @pl.when(i == _N3 - 1)
        def _emit():
            st_ref[...] = acc[...]

    cin = 1 if Din == 16 else 16
    xspec = pl.BlockSpec((BLK, Din), lambda i: (i, 0))
    wspec = pl.BlockSpec((cin, 16), lambda i: (0, 0))
    bspec = pl.BlockSpec((1, 16), lambda i: (0, 0))
    return pl.pallas_call(
        body,
        grid=(_N3,),
        in_specs=[xspec] * 4 + [wspec] * 4 + [bspec],
        out_specs=[pl.BlockSpec((OBLK, 64), lambda i: (i, 0)),
                   pl.BlockSpec((2, 64), lambda i: (0, 0))],
        out_shape=[jax.ShapeDtypeStruct((Vo, 64), jnp.float32),
                   jax.ShapeDtypeStruct((2, 64), jnp.float32)],
        scratch_shapes=[pltpu.VMEM((2, 64), jnp.float32)],
    )


@functools.lru_cache(maxsize=None)
def _make_bn_apply(Vo):
    BLK = Vo // _N3

    def body(y, st, gamma, beta, o_ref):
        s, sq = st[0:1, :], st[1:2, :]
        sc = s[:, :16] + s[:, 16:32] + s[:, 32:48] + s[:, 48:]
        sqc = sq[:, :16] + sq[:, 16:32] + sq[:, 32:48] + sq[:, 48:]
        cnt = 4.0 * Vo
        m = sc / cnt
        var = sqc / cnt - m * m
        inv = lax.rsqrt(var + 1e-5)
        mt = jnp.concatenate([m] * 4, axis=1)
        it = jnp.concatenate([inv] * 4, axis=1)
        o_ref[...] = gamma[...] * (y[...] - mt) * it + beta[...]

    return pl.pallas_call(
        body,
        grid=(_N3,),
        in_specs=[pl.BlockSpec((BLK, 64), lambda i: (i, 0)),
                  pl.BlockSpec((2, 64), lambda i: (0, 0)),
                  pl.BlockSpec((1, 64), lambda i: (0, 0)),
                  pl.BlockSpec((1, 64), lambda i: (0, 0))],
        out_specs=pl.BlockSpec((BLK, 64), lambda i: (i, 0)),
        out_shape=jax.ShapeDtypeStruct((Vo, 64), jnp.float32),
    )


def _dense_bn(V, Din, pool_dims, Xs, Wms, bias, gamma, beta):
    y, st = _make_dense(V, Din, pool_dims)(*Xs, *Wms, bias)
    return _make_bn_apply(y.shape[0])(y, st, gamma, beta)


@functools.lru_cache(maxsize=None)
def _make_final(V):
    def body(x0, x1, x2, x3, w0, w1, w2, w3, bias, o_ref):
        y = _combine((x0[...], x1[...], x2[...], x3[...]),
                     (w0[...], w1[...], w2[...], w3[...]), bias[...])
        m = jnp.max(y, axis=0, keepdims=True)
        rows = []
        for b in range(_B):
            sb = m[:, b * 10:(b + 1) * 10]
            mb = jnp.max(sb)
            rows.append(sb - mb - jnp.log(jnp.sum(jnp.exp(sb - mb))))
        o_ref[...] = jnp.concatenate(rows, axis=0)

    return pl.pallas_call(
        body, out_shape=jax.ShapeDtypeStruct((_B, 10), jnp.float32))


def kernel(x, rows1, cols1, vals1, rows2, cols2, vals2, rows3, cols3, vals3,
           W1, b1, W2, b2, W3, b3, W4, b4, W5, b5, W6, b6, gamma2, beta2,
           gamma3, beta3, gamma4, beta4, gamma5, beta5, gamma6, beta6):
    V = [_NX1[i] * _NX2[i] * _NX3[i] for i in range(3)]

    W1m = tuple(W1[k] for k in range(_K))
    Wm = {2: W2, 3: W3, 4: W4, 5: W5, 6: W6}
    Wm = {j: tuple(w[k] for k in range(_K)) for j, w in Wm.items()}
    biases = {j: b[None, :]
              for j, b in ((1, b1), (2, b2), (3, b3), (4, b4), (5, b5),
                           (6, b6))}
    gammas = {j: jnp.tile(g[None, :], (1, _B))
              for j, g in ((2, gamma2), (3, gamma3), (4, gamma4),
                           (5, gamma5), (6, gamma6))}
    betas = {j: jnp.tile(b[None, :], (1, _B))
             for j, b in ((2, beta2), (3, beta3), (4, beta4), (5, beta5),
                          (6, beta6))}

    X0 = jnp.pad(x.reshape(_B, V[0]).T, ((0, 0), (0, 12)))

    Xs = _cheb_states(X0, cols1, vals1, V[0], 16)
    h = _dense_bn(V[0], 16, None, Xs, W1m, biases[1], gammas[2], betas[2])
    Xs = _cheb_states(h, cols1, vals1, V[0], 64)
    h = _dense_bn(V[0], 64, (_NX3[0], _NX2[0], _NX1[0]), Xs, Wm[2],
                  biases[2], gammas[3], betas[3])
    Xs = _cheb_states(h, cols2, vals2, V[1], 64)
    h = _dense_bn(V[1], 64, None, Xs, Wm[3], biases[3], gammas[4], betas[4])
    Xs = _cheb_states(h, cols2, vals2, V[1], 64)
    h = _dense_bn(V[1], 64, (_NX3[1], _NX2[1], _NX1[1]), Xs, Wm[4],
                  biases[4], gammas[5], betas[5])
    Xs = _cheb_states(h, cols3, vals3, V[2], 64)
    h = _dense_bn(V[2], 64, None, Xs, Wm[5], biases[5], gammas[6], betas[6])
    Xs = _cheb_states(h, cols3, vals3, V[2], 64)
    return _make_final(V[2])(*Xs, *Wm[6], biases[6])


# speedup vs baseline: 329.6518x; 1.0016x over previous
import functools

import jax
import jax.numpy as jnp
from jax import lax
from jax.experimental import pallas as pl
from jax.experimental.pallas import tpu as pltpu
from jax.experimental.pallas import tpu_sc as plsc

_NX1 = [64, 32, 16]
_NX2 = [64, 32, 16]
_NX3 = [6, 6, 6]
_K = 4
_B = 4
_NC = 2
_NS = 16
_NW = _NC * _NS
_DEG = 16
_CN = 8

_GDN = lax.GatherDimensionNumbers(offset_dims=(), collapsed_slice_dims=(0,),
                                  start_index_map=(0,))


def _bcast_lane(v, k):
    idx = jnp.full((16, 1), k, jnp.int32)
    return lax.gather(v, idx, _GDN, (1,),
                      mode=lax.GatherScatterMode.PROMISE_IN_BOUNDS)


@functools.lru_cache(maxsize=None)
def _make_spmv(V, D, alpha, beta):
    NT = V // _NW
    if D == 16 and NT % 128 == 0:
        CN = 64
    else:
        CN = 32 if NT % 64 == 0 else 24
    G = CN * _DEG // 128
    NCH = NT // CN
    GT = NT // 8
    NJ = D // 16
    mesh = plsc.VectorSubcoreMesh(core_axis_name="c", subcore_axis_name="s",
                                  num_cores=_NC, num_subcores=_NS)

    @functools.partial(
        pl.kernel,
        out_type=jax.ShapeDtypeStruct((V, D), jnp.float32),
        mesh=mesh,
        compiler_params=pltpu.CompilerParams(use_tc_tiling_on_sc=False),
        scratch_types=[
            pltpu.VMEM((GT, 128), jnp.int32),
            pltpu.VMEM((NT * _DEG,), jnp.float32),
            [pltpu.VMEM((CN * _DEG, D), jnp.float32)] * 2,
            [pltpu.VMEM((CN, D), jnp.float32)] * 2,
            [pltpu.VMEM((CN, D), jnp.float32)] * 2,
            [pltpu.VMEM((CN, D), jnp.float32)] * 2,
            [pltpu.SemaphoreType.DMA] * 2,
            [pltpu.SemaphoreType.DMA] * 2,
            [pltpu.SemaphoreType.DMA] * 2,
        ],
    )
    def spmv(xc, xp, cols, vals, out, colv, valv, gath, xcv, xpv, outv,
             sg, sl, so):
        wid = lax.axis_index("s") * _NC + lax.axis_index("c")
        tile_base = wid * NT
        pltpu.sync_copy(cols.at[pl.ds(wid * GT, GT)], colv)
        pltpu.sync_copy(vals.at[pl.ds(tile_base * _DEG, NT * _DEG)], valv)

        def start(ci, b):
            nb = pl.multiple_of(tile_base + ci * CN, 8)
            for g in range(G):
                pltpu.async_copy(xc.at[colv.at[ci * G + g]],
                                 gath[b].at[pl.ds(128 * g, 128)], sg[b])
            pltpu.async_copy(xc.at[pl.ds(nb, CN)], xcv[b], sl[b])
            if beta != 0.0:
                pltpu.async_copy(xp.at[pl.ds(nb, CN)], xpv[b], sl[b])

        def wait_in(b):
            pltpu.make_async_copy(xc.at[colv.at[0]], gath[b], sg[b]).wait()
            pltpu.make_async_copy(xc.at[pl.ds(0, CN)], xcv[b], sl[b]).wait()
            if beta != 0.0:
                pltpu.make_async_copy(xp.at[pl.ds(0, CN)], xpv[b],
                                      sl[b]).wait()

        def drain_out(b):
            pltpu.make_async_copy(outv[b], out.at[pl.ds(0, CN)],
                                  so[b]).wait()

        def process(ci, b):
            nb = pl.multiple_of(tile_base + ci * CN, 8)
            wait_in(b)

            def node_body(n, c2):
                r0 = n * _DEG
                w16 = valv[pl.ds((ci * CN + n) * _DEG, 16)]
                accs = [jnp.zeros((16,), jnp.float32) for _ in range(NJ)]
                for k in range(_DEG):
                    wk = _bcast_lane(w16, k)
                    for j in range(NJ):
                        accs[j] = accs[j] + wk * gath[b][r0 + k,
                                                        pl.ds(16 * j, 16)]
                for j in range(NJ):
                    res = alpha * (accs[j] - xcv[b][n, pl.ds(16 * j, 16)])
                    if beta != 0.0:
                        res = res + beta * xpv[b][n, pl.ds(16 * j, 16)]
                    outv[b][n, pl.ds(16 * j, 16)] = res
                return c2

            lax.fori_loop(0, CN, node_body, 0)
            pltpu.async_copy(outv[b], out.at[pl.ds(nb, CN)], so[b])

        start(0, 0)
        start(1, 1)
        last = NCH - 1
        process(0, 0)
        start(jnp.minimum(2, last), 0)
        process(1, 1)
        start(jnp.minimum(3, last), 1)

        def outer_body(o, carry):
            for b in (0, 1):
                ci = o * 2 + b
                drain_out(b)
                process(ci, b)
                start(jnp.minimum(ci + 2, last), b)
            return carry

        lax.fori_loop(1, NCH // 2, outer_body, 0)
        for b in (0, 1):
            wait_in(b)
            drain_out(b)

    return spmv


def _cheb_states(X0, cols, vals, V, D):
    cols2d = cols.reshape(V * _DEG // 128, 128)
    sp_a = _make_spmv(V, D, 1.0, 0.0)
    sp_b = _make_spmv(V, D, 2.0, -1.0)
    X1 = sp_a(X0, X0, cols2d, vals)
    X2 = sp_b(X1, X0, cols2d, vals)
    X3 = sp_b(X2, X1, cols2d, vals)
    return X0, X1, X2, X3


def _combine(xs, ws, bias):
    cin = ws[0].shape[0]
    cols = []
    for b in range(_B):
        yb = bias
        for k in range(_K):
            if cin == 1:
                yb = yb + xs[k][:, b:b + 1] * ws[k]
            else:
                yb = yb + jnp.dot(xs[k][:, 16 * b:16 * b + 16], ws[k],
                                  preferred_element_type=jnp.float32)
        cols.append(yb)
    return jax.nn.relu(jnp.concatenate(cols, axis=1))


_N3 = 6


@functools.lru_cache(maxsize=None)
def _make_dense(V, Din, pool_dims):
    BLK = V // _N3
    pool = pool_dims is not None
    OBLK = BLK // 4 if pool else BLK
    Vo = V // 4 if pool else V

    def body(x0, x1, x2, x3, w0, w1, w2, w3, bias, y_ref, st_ref, acc):
        i = pl.program_id(0)
        y = _combine((x0[...], x1[...], x2[...], x3[...]),
                     (w0[...], w1[...], w2[...], w3[...]), bias[...])
        if pool:
            _, n2, n1 = pool_dims
            y = jnp.max(y.reshape(BLK // 2, 2, 64), axis=1)
            y = jnp.max(y.reshape(n2 // 2, 2, n1 // 2, 64), axis=1)
            y = y.reshape(OBLK, 64)
        y_ref[...] = y
        st = jnp.concatenate([jnp.sum(y, axis=0, keepdims=True),
                              jnp.sum(y * y, axis=0, keepdims=True)], axis=0)

        @pl.when(i == 0)
        def _init():
            acc[...] = st

        @pl.when(i > 0)
        def _accum():
            acc[...] = acc[...] + st

        @pl.when(i == _N3 - 1)
        def _emit():
            st_ref[...] = acc[...]

    cin = 1 if Din == 16 else 16
    xspec = pl.BlockSpec((BLK, Din), lambda i: (i, 0))
    wspec = pl.BlockSpec((cin, 16), lambda i: (0, 0))
    bspec = pl.BlockSpec((1, 16), lambda i: (0, 0))
    return pl.pallas_call(
        body,
        grid=(_N3,),
        in_specs=[xspec] * 4 + [wspec] * 4 + [bspec],
        out_specs=[pl.BlockSpec((OBLK, 64), lambda i: (i, 0)),
                   pl.BlockSpec((2, 64), lambda i: (0, 0))],
        out_shape=[jax.ShapeDtypeStruct((Vo, 64), jnp.float32),
                   jax.ShapeDtypeStruct((2, 64), jnp.float32)],
        scratch_shapes=[pltpu.VMEM((2, 64), jnp.float32)],
    )


@functools.lru_cache(maxsize=None)
def _make_bn_apply(Vo):
    BLK = Vo // _N3

    def body(y, st, gamma, beta, o_ref):
        s, sq = st[0:1, :], st[1:2, :]
        sc = s[:, :16] + s[:, 16:32] + s[:, 32:48] + s[:, 48:]
        sqc = sq[:, :16] + sq[:, 16:32] + sq[:, 32:48] + sq[:, 48:]
        cnt = 4.0 * Vo
        m = sc / cnt
        var = sqc / cnt - m * m
        inv = lax.rsqrt(var + 1e-5)
        mt = jnp.concatenate([m] * 4, axis=1)
        it = jnp.concatenate([inv] * 4, axis=1)
        o_ref[...] = gamma[...] * (y[...] - mt) * it + beta[...]

    return pl.pallas_call(
        body,
        grid=(_N3,),
        in_specs=[pl.BlockSpec((BLK, 64), lambda i: (i, 0)),
                  pl.BlockSpec((2, 64), lambda i: (0, 0)),
                  pl.BlockSpec((1, 64), lambda i: (0, 0)),
                  pl.BlockSpec((1, 64), lambda i: (0, 0))],
        out_specs=pl.BlockSpec((BLK, 64), lambda i: (i, 0)),
        out_shape=jax.ShapeDtypeStruct((Vo, 64), jnp.float32),
    )


def _dense_bn(V, Din, pool_dims, Xs, Wms, bias, gamma, beta):
    y, st = _make_dense(V, Din, pool_dims)(*Xs, *Wms, bias)
    return _make_bn_apply(y.shape[0])(y, st, gamma, beta)


@functools.lru_cache(maxsize=None)
def _make_final(V):
    def body(x0, x1, x2, x3, w0, w1, w2, w3, bias, o_ref):
        y = _combine((x0[...], x1[...], x2[...], x3[...]),
                     (w0[...], w1[...], w2[...], w3[...]), bias[...])
        m = jnp.max(y, axis=0, keepdims=True)
        rows = []
        for b in range(_B):
            sb = m[:, b * 10:(b + 1) * 10]
            mb = jnp.max(sb)
            rows.append(sb - mb - jnp.log(jnp.sum(jnp.exp(sb - mb))))
        o_ref[...] = jnp.concatenate(rows, axis=0)

    return pl.pallas_call(
        body, out_shape=jax.ShapeDtypeStruct((_B, 10), jnp.float32))


def kernel(x, rows1, cols1, vals1, rows2, cols2, vals2, rows3, cols3, vals3,
           W1, b1, W2, b2, W3, b3, W4, b4, W5, b5, W6, b6, gamma2, beta2,
           gamma3, beta3, gamma4, beta4, gamma5, beta5, gamma6, beta6):
    V = [_NX1[i] * _NX2[i] * _NX3[i] for i in range(3)]

    W1m = tuple(W1[k] for k in range(_K))
    Wm = {2: W2, 3: W3, 4: W4, 5: W5, 6: W6}
    Wm = {j: tuple(w[k] for k in range(_K)) for j, w in Wm.items()}
    biases = {j: b[None, :]
              for j, b in ((1, b1), (2, b2), (3, b3), (4, b4), (5, b5),
                           (6, b6))}
    gammas = {j: jnp.tile(g[None, :], (1, _B))
              for j, g in ((2, gamma2), (3, gamma3), (4, gamma4),
                           (5, gamma5), (6, gamma6))}
    betas = {j: jnp.tile(b[None, :], (1, _B))
             for j, b in ((2, beta2), (3, beta3), (4, beta4), (5, beta5),
                          (6, beta6))}

    X0 = jnp.pad(x.reshape(_B, V[0]).T, ((0, 0), (0, 12)))

    Xs = _cheb_states(X0, cols1, vals1, V[0], 16)
    h = _dense_bn(V[0], 16, None, Xs, W1m, biases[1], gammas[2], betas[2])
    Xs = _cheb_states(h, cols1, vals1, V[0], 64)
    h = _dense_bn(V[0], 64, (_NX3[0], _NX2[0], _NX1[0]), Xs, Wm[2],
                  biases[2], gammas[3], betas[3])
    Xs = _cheb_states(h, cols2, vals2, V[1], 64)
    h = _dense_bn(V[1], 64, None, Xs, Wm[3], biases[3], gammas[4], betas[4])
    Xs = _cheb_states(h, cols2, vals2, V[1], 64)
    h = _dense_bn(V[1], 64, (_NX3[1], _NX2[1], _NX1[1]), Xs, Wm[4],
                  biases[4], gammas[5], betas[5])
    Xs = _cheb_states(h, cols3, vals3, V[2], 64)
    h = _dense_bn(V[2], 64, None, Xs, Wm[5], biases[5], gammas[6], betas[6])
    Xs = _cheb_states(h, cols3, vals3, V[2], 64)
    return _make_final(V[2])(*Xs, *Wm[6], biases[6])
